# baseline (speedup 1.0000x reference)
import functools

import jax
import jax.numpy as jnp
from jax import lax
from jax.experimental import pallas as pl
from jax.experimental.pallas import tpu as pltpu
from jax.experimental.pallas import tpu_sc as plsc

_GW = 128
_DEPTH = 3


@functools.lru_cache(maxsize=None)
def _make_gather(N, D, NW):
    per_w = N // NW
    G = 8
    C = G * _GW
    n_chunks = per_w // C
    n_blocks = per_w // _GW
    mesh = plsc.VectorSubcoreMesh(core_axis_name="c", subcore_axis_name="s")

    @functools.partial(
        pl.kernel,
        mesh=mesh,
        out_type=jax.ShapeDtypeStruct((N, D), jnp.float32),
        compiler_params=pltpu.CompilerParams(use_tc_tiling_on_sc=False),
        scratch_types=[
            pltpu.VMEM((n_blocks, _GW), jnp.int32),
            pltpu.VMEM((_DEPTH, C, D), jnp.float32),
            [pltpu.SemaphoreType.DMA] * _DEPTH,
            [pltpu.SemaphoreType.DMA] * _DEPTH,
        ],
    )
    def k(idx_hbm, tab_hbm, out_hbm, idx_v, rows_v, gsem, osem):
        wid = lax.axis_index("s") * 2 + lax.axis_index("c")
        base = wid * per_w
        rbase = wid * n_blocks

        pltpu.sync_copy(idx_hbm.at[pl.ds(rbase, n_blocks)], idx_v)

        def fire_gathers(c):
            b = c % _DEPTH
            return [
                pltpu.async_copy(
                    tab_hbm.at[idx_v.at[c * G + g]],
                    rows_v.at[b].at[pl.ds(g * _GW, _GW)],
                    gsem[b],
                )
                for g in range(G)
            ]

        def fire_owrite(c):
            b = c % _DEPTH
            return pltpu.async_copy(
                rows_v.at[b], out_hbm.at[pl.ds(base + c * C, C)], osem[b]
            )

        gd, od = {}, {}
        for c in range(min(_DEPTH - 1, n_chunks)):
            gd[c] = fire_gathers(c)
        for c in range(n_chunks):
            nxt = c + _DEPTH - 1
            if nxt < n_chunks:
                if nxt - _DEPTH >= 0:
                    od.pop(nxt - _DEPTH).wait()
                gd[nxt] = fire_gathers(nxt)
            for d in gd.pop(c):
                d.wait()
            od[c] = fire_owrite(c)
        for c in sorted(od):
            od.pop(c).wait()

    return k


def kernel(input, tables):
    B, F = input.shape
    _, V, D = tables.shape
    N = B * F
    info = plsc.get_sparse_core_info()
    NW = info.num_cores * info.num_subcores
    flat_idx = (input + jnp.arange(F, dtype=input.dtype) * V).reshape(N // _GW, _GW)
    flat_tab = tables.reshape(F * V, D)
    out = _make_gather(N, D, NW)(flat_idx, flat_tab)
    out_t = jnp.transpose(out.reshape(B, F, D), (1, 2, 0))
    return out_t.reshape(F * D, B).T


# speedup vs baseline: 1.3271x; 1.3271x over previous
import functools

import jax
import jax.numpy as jnp
from jax import lax
from jax.experimental import pallas as pl
from jax.experimental.pallas import tpu as pltpu
from jax.experimental.pallas import tpu_sc as plsc

_GW = 128
_PITCH = 513


@functools.lru_cache(maxsize=None)
def _make_gather(F, V, D, B, NW):
    bpw = B // NW
    G = bpw // _GW
    n_groups = bpw // 16
    n_pairs = F // 2
    mesh = plsc.VectorSubcoreMesh(core_axis_name="c", subcore_axis_name="s")

    @functools.partial(
        pl.kernel,
        mesh=mesh,
        out_type=jax.ShapeDtypeStruct((F, D, B), jnp.float32),
        compiler_params=pltpu.CompilerParams(
            use_tc_tiling_on_sc=False, needs_layout_passes=False
        ),
        scratch_types=[
            pltpu.VMEM((2, G, _GW), jnp.int32),
            pltpu.VMEM((2, bpw, D), jnp.float32),
            pltpu.VMEM((2, D * _PITCH), jnp.float32),
            pltpu.VMEM((2, D, bpw), jnp.float32),
            [pltpu.SemaphoreType.DMA] * 2,
            [pltpu.SemaphoreType.DMA] * 2,
        ],
    )
    def k(idx_hbm, tab_hbm, out_hbm, fid_v, rows_v, trans_v, pack_v,
          gsem, osem):
        wid = lax.axis_index("s") * 2 + lax.axis_index("c")
        b0 = wid * bpw
        i16 = lax.iota(jnp.int32, 16)
        sc_lo = i16 * _PITCH
        sc_hi = (i16 + 16) * _PITCH

        def fire(f, par):
            pltpu.sync_copy(idx_hbm.at[f].at[pl.ds(wid * G, G)],
                            fid_v.at[par])
            off = f * V

            def addb(j, carry):
                def addl(l, c2):
                    s = pl.ds(l * 16, 16)
                    fid_v[par, j, s] = fid_v[par, j, s] + off
                    return c2
                return lax.fori_loop(0, _GW // 16, addl, carry)

            lax.fori_loop(0, G, addb, 0)
            for j in range(G):
                pltpu.async_copy(
                    tab_hbm.at[fid_v.at[par].at[j]],
                    rows_v.at[par].at[pl.ds(j * _GW, _GW)],
                    gsem[par],
                )

        def wait_gathers(par):
            for j in range(G):
                pltpu.make_async_copy(
                    tab_hbm.at[fid_v.at[par].at[j]],
                    rows_v.at[par].at[pl.ds(j * _GW, _GW)],
                    gsem[par],
                ).wait()

        def transpose(par):
            def body(g, carry):
                for j in range(16):
                    b = g * 16 + j
                    plsc.store_scatter(trans_v.at[par], [sc_lo + b],
                                       rows_v[par, b, pl.ds(0, 16)])
                    plsc.store_scatter(trans_v.at[par], [sc_hi + b],
                                       rows_v[par, b, pl.ds(16, 16)])
                return carry
            lax.fori_loop(0, n_groups, body, 0)

        def compact(par):
            def body(d, carry):
                def inner(c, c2):
                    src = pl.ds(d * _PITCH + c * 16, 16)
                    pack_v[par, d, pl.ds(c * 16, 16)] = trans_v[par, src]
                    return c2
                return lax.fori_loop(0, bpw // 16, inner, carry)
            lax.fori_loop(0, D, body, 0)

        def fire_owrite(f, par):
            pltpu.async_copy(
                pack_v.at[par],
                out_hbm.at[f].at[:, pl.ds(b0, bpw)],
                osem[par],
            )

        def wait_owrite(par):
            pltpu.make_async_copy(
                pack_v.at[par],
                out_hbm.at[0].at[:, pl.ds(b0, bpw)],
                osem[par],
            ).wait()

        def process(f, par, *, first, last):
            wait_gathers(par)
            if not first:
                wait_owrite(par)
            transpose(par)
            if not last:
                @pl.when(f + 2 < F)
                def _():
                    fire(f + 2, par)
            compact(par)
            fire_owrite(f, par)

        fire(0, 0)
        fire(1, 1)

        def pair(t, carry):
            f0 = 2 * t
            process(f0, 0, first=False, last=False)
            process(f0 + 1, 1, first=False, last=False)
            return carry

        process(0, 0, first=True, last=False)
        process(1, 1, first=True, last=False)
        lax.fori_loop(1, n_pairs, pair, 0)
        wait_owrite(0)
        wait_owrite(1)

    return k


def kernel(input, tables):
    B, F = input.shape
    _, V, D = tables.shape
    info = plsc.get_sparse_core_info()
    NW = info.num_cores * info.num_subcores
    idx3 = input.T.reshape(F, B // _GW, _GW)
    flat_tab = tables.reshape(F * V, D)
    out3 = _make_gather(F, V, D, B, NW)(idx3, flat_tab)
    return out3.reshape(F * D, B).T


# speedup vs baseline: 1.3814x; 1.0409x over previous
import functools

import jax
import jax.numpy as jnp
from jax import lax
from jax.experimental import pallas as pl
from jax.experimental.pallas import tpu as pltpu
from jax.experimental.pallas import tpu_sc as plsc

_GW = 128
_PITCH = 513


@functools.lru_cache(maxsize=None)
def _make_gather(F, V, D, B, NW):
    bpw = B // NW
    G = bpw // _GW
    n_groups = bpw // 16
    n_pairs = F // 2
    mesh = plsc.VectorSubcoreMesh(core_axis_name="c", subcore_axis_name="s")

    @functools.partial(
        pl.kernel,
        mesh=mesh,
        out_type=jax.ShapeDtypeStruct((F, D, B), jnp.float32),
        compiler_params=pltpu.CompilerParams(
            use_tc_tiling_on_sc=False, needs_layout_passes=False
        ),
        scratch_types=[
            pltpu.VMEM((F * G, _GW), jnp.int32),
            pltpu.VMEM((2, bpw, D), jnp.float32),
            pltpu.VMEM((2, D * _PITCH), jnp.float32),
            pltpu.VMEM((2, D, bpw), jnp.float32),
            [pltpu.SemaphoreType.DMA] * 2,
            [pltpu.SemaphoreType.DMA] * 2,
        ],
    )
    def k(idx_hbm, tab_hbm, out_hbm, fid_v, rows_v, trans_v, pack_v,
          gsem, osem):
        wid = lax.axis_index("s") * 2 + lax.axis_index("c")
        b0 = wid * bpw
        i16 = lax.iota(jnp.int32, 16)
        sc_lo = i16 * _PITCH
        sc_hi = (i16 + 16) * _PITCH

        pltpu.sync_copy(idx_hbm.at[wid], fid_v)

        def fire(f, par):
            for j in range(G):
                pltpu.async_copy(
                    tab_hbm.at[fid_v.at[f * G + j]],
                    rows_v.at[par].at[pl.ds(j * _GW, _GW)],
                    gsem[par],
                )

        def wait_gathers(par):
            for j in range(G):
                pltpu.make_async_copy(
                    tab_hbm.at[fid_v.at[j]],
                    rows_v.at[par].at[pl.ds(j * _GW, _GW)],
                    gsem[par],
                ).wait()

        def transpose(par):
            def body(g, carry):
                for j in range(16):
                    b = g * 16 + j
                    plsc.store_scatter(trans_v.at[par], [sc_lo + b],
                                       rows_v[par, b, pl.ds(0, 16)])
                    plsc.store_scatter(trans_v.at[par], [sc_hi + b],
                                       rows_v[par, b, pl.ds(16, 16)])
                return carry
            lax.fori_loop(0, n_groups, body, 0, unroll=2)

        def compact(par):
            def body(d, carry):
                for c in range(bpw // 16):
                    src = pl.ds(d * _PITCH + c * 16, 16)
                    pack_v[par, d, pl.ds(c * 16, 16)] = trans_v[par, src]
                return carry
            lax.fori_loop(0, D, body, 0, unroll=2)

        def fire_owrite(f, par):
            pltpu.async_copy(
                pack_v.at[par],
                out_hbm.at[f].at[:, pl.ds(b0, bpw)],
                osem[par],
            )

        def wait_owrite(par):
            pltpu.make_async_copy(
                pack_v.at[par],
                out_hbm.at[0].at[:, pl.ds(b0, bpw)],
                osem[par],
            ).wait()

        def process(f, par, *, first, last):
            wait_gathers(par)
            if not first:
                wait_owrite(par)
            transpose(par)
            if not last:
                @pl.when(f + 2 < F)
                def _():
                    fire(f + 2, par)
            compact(par)
            fire_owrite(f, par)

        fire(0, 0)
        fire(1, 1)

        def pair(t, carry):
            f0 = 2 * t
            process(f0, 0, first=False, last=False)
            process(f0 + 1, 1, first=False, last=False)
            return carry

        process(0, 0, first=True, last=False)
        process(1, 1, first=True, last=False)
        lax.fori_loop(1, n_pairs, pair, 0)
        wait_owrite(0)
        wait_owrite(1)

    return k


def kernel(input, tables):
    B, F = input.shape
    _, V, D = tables.shape
    info = plsc.get_sparse_core_info()
    NW = info.num_cores * info.num_subcores
    G = (B // NW) // _GW
    fids = (input + jnp.arange(F, dtype=input.dtype) * V).T
    ids4 = jnp.transpose(
        fids.reshape(F, NW, G, _GW), (1, 0, 2, 3)
    ).reshape(NW, F * G, _GW)
    flat_tab = tables.reshape(F * V, D)
    out3 = _make_gather(F, V, D, B, NW)(ids4, flat_tab)
    return out3.reshape(F * D, B).T


# speedup vs baseline: 1.4202x; 1.0281x over previous
import functools

import jax
import jax.numpy as jnp
from jax import lax
from jax.experimental import pallas as pl
from jax.experimental.pallas import tpu as pltpu
from jax.experimental.pallas import tpu_sc as plsc

_GW = 128
_BC = 256
_PITCH = 257


@functools.lru_cache(maxsize=None)
def _make_gather(F, V, D, B, NW):
    bpw = B // NW
    n_chunks = bpw // _BC
    G = _BC // _GW
    n_items = F * n_chunks
    Q = 128 // D
    mesh = plsc.VectorSubcoreMesh(core_axis_name="c", subcore_axis_name="s")

    @functools.partial(
        pl.kernel,
        mesh=mesh,
        out_type=jax.ShapeDtypeStruct((F * D, B), jnp.float32),
        compiler_params=pltpu.CompilerParams(
            use_tc_tiling_on_sc=True, needs_layout_passes=False
        ),
        scratch_types=[
            pltpu.VMEM((F * 2 * G, _GW), jnp.int32),
            pltpu.VMEM((2, G, _GW), jnp.int32),
            pltpu.VMEM((2, _BC, 128), jnp.float32),
            pltpu.VMEM((8320,), jnp.float32),
            pltpu.VMEM((8320,), jnp.float32),
            pltpu.VMEM((D, _BC), jnp.float32),
            pltpu.VMEM((D, _BC), jnp.float32),
            [pltpu.SemaphoreType.DMA] * 2,
            [pltpu.SemaphoreType.DMA] * 2,
        ],
    )
    def k(idx_hbm, tab_hbm, out_hbm, ids_v, qid_v, rows_v, trans_0,
          trans_1, pack_0, pack_1, gsem, osem):
        trans_b = (trans_0, trans_1)
        pack_b = (pack_0, pack_1)
        wid = lax.axis_index("s") * 2 + lax.axis_index("c")
        i16 = lax.iota(jnp.int32, 16)
        sc_lo = i16 * _PITCH
        sc_hi = (i16 + 16) * _PITCH

        n_blk = n_items * G
        pltpu.sync_copy(idx_hbm.at[pl.ds(wid * n_blk, n_blk)], ids_v)

        def fire(it, par):
            f = it // n_chunks
            base = it * G
            foff = f * (V // Q)
            for j in range(G):
                for l in range(_GW // 16):
                    s = pl.ds(l * 16, 16)
                    qid_v[par, j, s] = (
                        lax.shift_right_logical(ids_v[base + j, s], 2) + foff
                    )
            for j in range(G):
                pltpu.async_copy(
                    tab_hbm.at[qid_v.at[par].at[j]],
                    rows_v.at[par].at[pl.ds(j * _GW, _GW)],
                    gsem[par],
                )

        def wait_gathers(par):
            for j in range(G):
                pltpu.make_async_copy(
                    tab_hbm.at[qid_v.at[par].at[j]],
                    rows_v.at[par].at[pl.ds(j * _GW, _GW)],
                    gsem[par],
                ).wait()

        def extract(it, par):
            base = it * G

            def body(g, carry):
                bi = lax.shift_right_logical(g, 3)
                bs = (g & 7) * 16
                idvec = ids_v[base + bi, pl.ds(bs, 16)]
                offv = (idvec & (Q - 1)) * D
                for j in range(16):
                    b = g * 16 + j
                    off = offv[j]
                    plsc.store_scatter(
                        trans_b[par], [sc_lo + b],
                        rows_v[par, b, pl.ds(off, 16)])
                    plsc.store_scatter(
                        trans_b[par], [sc_hi + b],
                        rows_v[par, b, pl.ds(off + 16, 16)])
                return carry
            lax.fori_loop(0, _BC // 16, body, 0, unroll=2)

        def compact(par):
            def body(d, carry):
                for c in range(_BC // 16):
                    src = pl.ds(d * _PITCH + c * 16, 16)
                    pack_b[par][d, pl.ds(c * 16, 16)] = trans_b[par][src]
                return carry
            lax.fori_loop(0, D, body, 0, unroll=2)

        def fire_owrite(it, par):
            f = it // n_chunks
            c = lax.rem(it, n_chunks)
            b0 = wid * bpw + c * _BC
            pltpu.async_copy(
                pack_b[par],
                out_hbm.at[pl.ds(f * D, D), pl.ds(b0, _BC)],
                osem[par],
            )

        def wait_owrite(par):
            pltpu.make_async_copy(
                pack_b[par],
                out_hbm.at[pl.ds(0, D), pl.ds(wid * bpw, _BC)],
                osem[par],
            ).wait()

        def process(it, par, *, first):
            wait_gathers(par)
            if not first:
                wait_owrite(par)
            extract(it, par)
            @pl.when(it + 2 < n_items)
            def _():
                fire(it + 2, par)
            compact(par)
            fire_owrite(it, par)

        fire(0, 0)
        fire(1, 1)

        def pair(t, carry):
            i0 = 2 * t
            process(i0, 0, first=False)
            process(i0 + 1, 1, first=False)
            return carry

        process(0, 0, first=True)
        process(1, 1, first=True)
        lax.fori_loop(1, n_items // 2, pair, 0)
        wait_owrite(0)
        wait_owrite(1)

    return k


def kernel(input, tables):
    B, F = input.shape
    _, V, D = tables.shape
    info = plsc.get_sparse_core_info()
    NW = info.num_cores * info.num_subcores
    g_tot = (B // NW) // _GW
    ids2 = jnp.transpose(
        input.T.reshape(F, NW, g_tot, _GW), (1, 0, 2, 3)
    ).reshape(NW * F * g_tot, _GW)
    tab2 = tables.reshape(F * V * D // 128, 128)
    out2 = _make_gather(F, V, D, B, NW)(ids2, tab2)
    return out2.T


# speedup vs baseline: 1.4893x; 1.0487x over previous
import functools

import jax
import jax.numpy as jnp
from jax import lax
from jax.experimental import pallas as pl
from jax.experimental.pallas import tpu as pltpu
from jax.experimental.pallas import tpu_sc as plsc

_GW = 128
_DEPTH = 3


@functools.lru_cache(maxsize=None)
def _make_gather(N, D, NW):
    per_w = N // NW
    G = 8
    C = G * _GW
    n_chunks = per_w // C
    n_blocks = per_w // _GW
    mesh = plsc.VectorSubcoreMesh(core_axis_name="c", subcore_axis_name="s")

    @functools.partial(
        pl.kernel,
        mesh=mesh,
        out_type=jax.ShapeDtypeStruct((N, D), jnp.float32),
        compiler_params=pltpu.CompilerParams(use_tc_tiling_on_sc=False),
        scratch_types=[
            pltpu.VMEM((n_blocks, _GW), jnp.int32),
            pltpu.VMEM((_DEPTH, C, D), jnp.float32),
            [pltpu.SemaphoreType.DMA] * _DEPTH,
            [pltpu.SemaphoreType.DMA] * _DEPTH,
        ],
    )
    def k(idx_hbm, tab_hbm, out_hbm, idx_v, rows_v, gsem, osem):
        wid = lax.axis_index("s") * 2 + lax.axis_index("c")
        base = wid * per_w
        rbase = wid * n_blocks

        pltpu.sync_copy(idx_hbm.at[pl.ds(rbase, n_blocks)], idx_v)

        def fire_gathers(c):
            b = c % _DEPTH
            return [
                pltpu.async_copy(
                    tab_hbm.at[idx_v.at[c * G + g]],
                    rows_v.at[b].at[pl.ds(g * _GW, _GW)],
                    gsem[b],
                )
                for g in range(G)
            ]

        def fire_owrite(c):
            b = c % _DEPTH
            return pltpu.async_copy(
                rows_v.at[b], out_hbm.at[pl.ds(base + c * C, C)], osem[b]
            )

        gd, od = {}, {}
        for c in range(min(_DEPTH - 1, n_chunks)):
            gd[c] = fire_gathers(c)
        for c in range(n_chunks):
            nxt = c + _DEPTH - 1
            if nxt < n_chunks:
                if nxt - _DEPTH >= 0:
                    od.pop(nxt - _DEPTH).wait()
                gd[nxt] = fire_gathers(nxt)
            for d in gd.pop(c):
                d.wait()
            od[c] = fire_owrite(c)
        for c in sorted(od):
            od.pop(c).wait()

    return k


def kernel(input, tables):
    B, F = input.shape
    _, V, D = tables.shape
    N = B * F
    info = plsc.get_sparse_core_info()
    NW = info.num_cores * info.num_subcores
    flat_idx = (input + jnp.arange(F, dtype=input.dtype) * V).reshape(N // _GW, _GW)
    flat_tab = tables.reshape(F * V, D)
    out = _make_gather(N, D, NW)(flat_idx, flat_tab)
    return out.reshape(B, F * D)
